# Initial kernel scaffold; baseline (speedup 1.0000x reference)
import jax
import jax.numpy as jnp
from jax.experimental import pallas as pl


def kernel(x, W_lin, b_lin, W_emb, W1, b1, W2, b2, W3, b3):
    raise NotImplementedError("write your pallas kernel here")


# speedup vs baseline: 2.3403x; 2.3403x over previous
import jax
import jax.numpy as jnp
from jax import lax
from jax.experimental import pallas as pl
from jax.experimental.pallas import tpu as pltpu
from jax.experimental.pallas import tpu_sc as plsc

B = 16384
F = 26
D = 16
L = 16
NC, NS = 2, 16
NW = NC * NS
E_W = B // NW
EC = 4
IC = F * EC
NCHUNK = E_W // EC
FIELD_SIZE = 100000
TOTAL = F * FIELD_SIZE

NBUF = 4

_SC_MESH = plsc.VectorSubcoreMesh(core_axis_name="c", subcore_axis_name="s",
                                  num_cores=NC, num_subcores=NS)
_SC_PARAMS = pltpu.CompilerParams(needs_layout_passes=False,
                                  use_tc_tiling_on_sc=False)


def _worker_id():
    return lax.axis_index("s") * NC + lax.axis_index("c")


def _sc_emb_body(midx_hbm, emb_hbm, bi_hbm, midx_v, rows_v, sq_v, bi_v,
                 sem_i, sem_e):
    wid = _worker_id()
    pltpu.async_copy(midx_hbm.at[wid], midx_v, sem_i).wait()

    def issue(c, b):
        pltpu.make_async_copy(emb_hbm.at[midx_v.at[c]], rows_v.at[b],
                              sem_e.at[b]).start()

    for b in range(NBUF):
        issue(b, b)

    def outer_body(i, carry):
        for b in range(NBUF):
            c = i * NBUF + b
            pltpu.make_async_copy(emb_hbm.at[midx_v.at[c]], rows_v.at[b],
                                  sem_e.at[b]).wait()
            for k in range(EC):
                r = rows_v[b, k * F]
                s = r
                q = r * r
                for j in range(1, F):
                    r = rows_v[b, k * F + j]
                    s = s + r
                    q = q + r * r
                sq_v[pl.ds((c * EC + k) * D, D)] = s * s - q
            nxt = c + NBUF

            @pl.when(nxt < NCHUNK)
            def _():
                issue(nxt, b)
        return carry

    lax.fori_loop(0, NCHUNK // NBUF, outer_body, 0)

    iota = lax.iota(jnp.int32, L)

    def group_body(g, carry):
        rbase = (g * L + iota) * D
        acc = jnp.zeros((L,), jnp.float32)
        for d in range(D):
            acc = acc + plsc.load_gather(sq_v, [rbase + d])
        bi_v[pl.ds(g * L, L)] = 0.5 * acc
        return carry

    lax.fori_loop(0, E_W // L, group_body, 0)

    pltpu.sync_copy(bi_v, bi_hbm.at[pl.ds(wid * E_W, E_W)])


_sc_emb_call = pl.kernel(
    _sc_emb_body,
    out_type=jax.ShapeDtypeStruct((B,), jnp.float32),
    mesh=_SC_MESH,
    compiler_params=_SC_PARAMS,
    scratch_types=[
        pltpu.VMEM((NCHUNK, IC), jnp.int32),
        pltpu.VMEM((NBUF, IC, D), jnp.float32),
        pltpu.VMEM((E_W * D,), jnp.float32),
        pltpu.VMEM((E_W,), jnp.float32),
        pltpu.SemaphoreType.DMA,
        pltpu.SemaphoreType.DMA((NBUF,)),
    ],
)


def _sc_lin_body(idx_hbm, lin_hbm, lo_hbm, idx_v, lin_v, lo_v, sem_i, sem_l):
    wid = _worker_id()
    pltpu.async_copy(idx_hbm.at[wid], idx_v, sem_i).wait()

    def issue(c, b):
        pltpu.make_async_copy(lin_hbm.at[idx_v.at[c]],
                              lin_v.at[pl.ds(c * IC, IC)],
                              sem_l.at[b]).start()

    for b in range(NBUF):
        issue(b, b)

    def outer_body(i, carry):
        for b in range(NBUF):
            c = i * NBUF + b
            pltpu.make_async_copy(lin_hbm.at[idx_v.at[c]],
                                  lin_v.at[pl.ds(c * IC, IC)],
                                  sem_l.at[b]).wait()
            nxt = c + NBUF

            @pl.when(nxt < NCHUNK)
            def _():
                issue(nxt, b)
        return carry

    lax.fori_loop(0, NCHUNK // NBUF, outer_body, 0)

    iota = lax.iota(jnp.int32, L)

    def group_body(g, carry):
        base = (g * L + iota) * F
        lacc = jnp.zeros((L,), jnp.float32)
        for j in range(F):
            lacc = lacc + plsc.load_gather(lin_v, [base + j])
        lo_v[pl.ds(g * L, L)] = lacc
        return carry

    lax.fori_loop(0, E_W // L, group_body, 0)

    pltpu.sync_copy(lo_v, lo_hbm.at[pl.ds(wid * E_W, E_W)])


_sc_lin_call = pl.kernel(
    _sc_lin_body,
    out_type=jax.ShapeDtypeStruct((B,), jnp.float32),
    mesh=_SC_MESH,
    compiler_params=_SC_PARAMS,
    scratch_types=[
        pltpu.VMEM((NCHUNK, IC), jnp.int32),
        pltpu.VMEM((E_W * F,), jnp.float32),
        pltpu.VMEM((E_W,), jnp.float32),
        pltpu.SemaphoreType.DMA,
        pltpu.SemaphoreType.DMA((NBUF,)),
    ],
)


TRW = 16384
TRK = 8 * TRW
TR_GRID = -(-TOTAL // TRK)
TOTAL2 = TR_GRID * TRK


def _tr_body(in_ref, out_ref):
    x = in_ref[...]
    stacked = jnp.concatenate(
        [x[:, j * TRW:(j + 1) * TRW] for j in range(8)], axis=0)
    out_ref[...] = stacked.T


_tr_call = pl.pallas_call(
    _tr_body,
    grid=(TR_GRID,),
    in_specs=[pl.BlockSpec((D, TRK), lambda i: (0, i))],
    out_specs=pl.BlockSpec((TRW, 8 * D), lambda i: (i, 0)),
    out_shape=jax.ShapeDtypeStruct((TOTAL2 // 8, 8 * D), jnp.float32),
)


BLK = 2048


def _mlp_body(bi_ref, lo_ref, w1_ref, b1_ref, w2_ref, b2_ref, w3_ref,
              bb_ref, out_ref):
    c1 = jnp.sum(w1_ref[...], axis=0, keepdims=True)
    x1 = jnp.maximum(bi_ref[...] * c1 + b1_ref[...], 0.0)
    h2 = jnp.maximum(
        jnp.dot(x1, w2_ref[...], preferred_element_type=jnp.float32)
        + b2_ref[...], 0.0)
    out = jnp.dot(h2, w3_ref[...], preferred_element_type=jnp.float32)
    out_ref[...] = out + lo_ref[...] + bb_ref[...]


_mlp_call = pl.pallas_call(
    _mlp_body,
    grid=(B // BLK,),
    in_specs=[
        pl.BlockSpec((BLK, 1), lambda i: (i, 0)),
        pl.BlockSpec((BLK, 1), lambda i: (i, 0)),
        pl.BlockSpec((D, 64), lambda i: (0, 0)),
        pl.BlockSpec((1, 64), lambda i: (0, 0)),
        pl.BlockSpec((64, 32), lambda i: (0, 0)),
        pl.BlockSpec((1, 32), lambda i: (0, 0)),
        pl.BlockSpec((32, 1), lambda i: (0, 0)),
        pl.BlockSpec((1, 1), lambda i: (0, 0)),
    ],
    out_specs=pl.BlockSpec((BLK, 1), lambda i: (i, 0)),
    out_shape=jax.ShapeDtypeStruct((B, 1), jnp.float32),
)


def kernel(x, W_lin, b_lin, W_emb, W1, b1, W2, b2, W3, b3):
    offs = jnp.arange(F, dtype=jnp.int32) * jnp.int32(FIELD_SIZE)
    idx = x.astype(jnp.int32) + offs[None, :]
    midx = ((idx & ~jnp.int32(TRK - 1))
            | ((idx & jnp.int32(TRW - 1)) << 3)
            | ((idx >> jnp.int32(TRW.bit_length() - 1)) & jnp.int32(7)))
    midx_r = midx.reshape(NW, NCHUNK, IC)
    idx_r = idx.reshape(NW, NCHUNK, IC)
    lo = _sc_lin_call(idx_r, W_lin.reshape(-1))
    emb_rm = _tr_call(jnp.transpose(W_emb)).reshape(TOTAL2, D)
    bi = _sc_emb_call(midx_r, emb_rm)
    bb = (b_lin + b3).reshape(1, 1)
    out = _mlp_call(bi.reshape(B, 1), lo.reshape(B, 1), W1,
                    b1.reshape(1, 64), W2, b2.reshape(1, 32), W3, bb)
    return out.reshape(B)
